# Initial kernel scaffold; baseline (speedup 1.0000x reference)
import jax
import jax.numpy as jnp
from jax.experimental import pallas as pl


def kernel(adv_pc, ori_pc):
    raise NotImplementedError("write your pallas kernel here")


# speedup vs baseline: 1.1996x; 1.1996x over previous
import functools

import jax
import jax.numpy as jnp
from jax.experimental import pallas as pl

_NQ = 16384
_NK = 8192
_QBLK = 1024
_LOSS_WEIGHT = 1.0


def _nn_body(q_ref, kt_ref, out_ref):
    kt = kt_ref[...]
    bias = -0.5 * jnp.sum(kt * kt, axis=0, keepdims=True)
    val = jnp.dot(q_ref[...], kt,
                  preferred_element_type=jnp.float32) + bias
    m = jnp.max(val, axis=1, keepdims=True)
    ids = jax.lax.broadcasted_iota(jnp.int32, val.shape, 1)
    idx = jnp.min(jnp.where(val == m, ids, _NK), axis=1)
    out_ref[0, 0, :] = jnp.broadcast_to(
        jnp.sum(idx.astype(jnp.float32)), (128,))


@functools.partial(jax.jit, static_argnames=())
def _nn_sums(q_pad, kt_pad):
    grid = _NQ // _QBLK
    return pl.pallas_call(
        _nn_body,
        grid=(grid,),
        in_specs=[
            pl.BlockSpec((_QBLK, 8), lambda i: (i, 0)),
            pl.BlockSpec((8, _NK), lambda i: (0, 0)),
        ],
        out_specs=pl.BlockSpec((1, 1, 128), lambda i: (i, 0, 0)),
        out_shape=jax.ShapeDtypeStruct((grid, 1, 128), jnp.float32),
    )(q_pad, kt_pad)


def kernel(adv_pc, ori_pc):
    q = jnp.concatenate([adv_pc[:, :3], ori_pc[:, :3]], axis=0)
    q_pad = jnp.pad(q, ((0, 0), (0, 5)))
    kt_pad = jnp.pad(ori_pc[:, :3].T, ((0, 5), (0, 0)))
    sums = _nn_sums(q_pad, kt_pad)
    return (jnp.sum(sums[:, 0, 0]) / jnp.float32(_NK)) * _LOSS_WEIGHT


# speedup vs baseline: 1.3851x; 1.1546x over previous
import functools

import jax
import jax.numpy as jnp
from jax.experimental import pallas as pl

_NQ = 16384
_NK = 8192
_QBLK = 1024
_LOSS_WEIGHT = 1.0


def _nn_body(q_ref, kt_ref, out_ref):
    val = jnp.dot(q_ref[...], kt_ref[...],
                  preferred_element_type=jnp.float32)
    m = jnp.max(val, axis=1, keepdims=True)
    ids = jax.lax.broadcasted_iota(jnp.int32, val.shape, 1)
    idx = jnp.min(jnp.where(val == m, ids, _NK), axis=1)
    out_ref[0, 0, :] = jnp.broadcast_to(
        jnp.sum(idx.astype(jnp.float32)), (128,))


@functools.partial(jax.jit, static_argnames=())
def _nn_sums(q_pad, kt_pad):
    grid = _NQ // _QBLK
    return pl.pallas_call(
        _nn_body,
        grid=(grid,),
        in_specs=[
            pl.BlockSpec((_QBLK, 8), lambda i: (i, 0)),
            pl.BlockSpec((8, _NK), lambda i: (0, 0)),
        ],
        out_specs=pl.BlockSpec((1, 1, 128), lambda i: (i, 0, 0)),
        out_shape=jax.ShapeDtypeStruct((grid, 1, 128), jnp.float32),
    )(q_pad, kt_pad)


def kernel(adv_pc, ori_pc):
    q = jnp.concatenate([adv_pc[:, :3], ori_pc[:, :3]], axis=0)
    ones = jnp.ones((_NQ, 1), jnp.float32)
    q_pad = jnp.pad(jnp.concatenate([q, ones], axis=1),
                    ((0, 0), (0, 4)))
    k = ori_pc[:, :3]
    bias = -0.5 * jnp.sum(k * k, axis=1, keepdims=True)
    kt_pad = jnp.pad(jnp.concatenate([k, bias], axis=1).T,
                     ((0, 4), (0, 0)))
    sums = _nn_sums(q_pad, kt_pad)
    return (jnp.sum(sums[:, 0, 0]) / jnp.float32(_NK)) * _LOSS_WEIGHT
